# Initial kernel scaffold; baseline (speedup 1.0000x reference)
import jax
import jax.numpy as jnp
from jax.experimental import pallas as pl


def kernel(h_src, h_dst, edge_index):
    raise NotImplementedError("write your pallas kernel here")


# speedup vs baseline: 2.6214x; 2.6214x over previous
import functools

import jax
import jax.numpy as jnp
from jax import lax
from jax.experimental import pallas as pl
from jax.experimental.pallas import tpu as pltpu
from jax.experimental.pallas import tpu_sc as plsc

L = 16


@functools.cache
def _make_sc_kernel(E, N, D):
    NW = 32
    per_w = E // NW
    C = 80
    n_chunks = per_w // C
    assert per_w % C == 0 and C % L == 0 and D % L == 0

    mesh = plsc.VectorSubcoreMesh(core_axis_name="c", subcore_axis_name="s")

    @functools.partial(
        pl.kernel,
        mesh=mesh,
        out_type=jax.ShapeDtypeStruct((E,), jnp.float32),
        compiler_params=pltpu.CompilerParams(needs_layout_passes=False),
        scratch_types=[
            pltpu.VMEM((C,), jnp.int32),
            pltpu.VMEM((C,), jnp.int32),
            pltpu.VMEM((C, D), jnp.float32),
            pltpu.VMEM((C, D), jnp.float32),
            pltpu.VMEM((C,), jnp.float32),
            pltpu.SemaphoreType.DMA,
            pltpu.SemaphoreType.DMA,
        ],
    )
    def sc_kernel(hsrc_hbm, hdst_hbm, sidx_hbm, didx_hbm, out_hbm,
                  sidx_v, didx_v, srow_v, drow_v, score_v, sem1, sem2):
        wid = lax.axis_index("s") * 2 + lax.axis_index("c")
        wbase = wid * per_w

        def chunk_body(ci, carry):
            base = wbase + ci * C
            pltpu.sync_copy(sidx_hbm.at[pl.ds(base, C)], sidx_v)
            pltpu.sync_copy(didx_hbm.at[pl.ds(base, C)], didx_v)
            cp1 = pltpu.async_copy(hsrc_hbm.at[sidx_v], srow_v, sem1)
            cp2 = pltpu.async_copy(hdst_hbm.at[didx_v], drow_v, sem2)
            cp1.wait()
            cp2.wait()
            lane = lax.iota(jnp.int32, L)

            def gbody(g, carry2):
                scores = jnp.zeros((L,), jnp.float32)
                for e16 in range(L):
                    e = g * L + e16
                    acc = srow_v[e, pl.ds(0, L)] * drow_v[e, pl.ds(0, L)]
                    for j in range(1, D // L):
                        acc += (srow_v[e, pl.ds(j * L, L)]
                                * drow_v[e, pl.ds(j * L, L)])
                    scores = jnp.where(lane == e16, jnp.sum(acc), scores)
                score_v[pl.ds(g * L, L)] = scores
                return carry2

            lax.fori_loop(0, C // L, gbody, 0)
            pltpu.sync_copy(score_v, out_hbm.at[pl.ds(base, C)])
            return carry

        lax.fori_loop(0, n_chunks, chunk_body, 0)

    return sc_kernel


def kernel(h_src, h_dst, edge_index):
    N, D = h_src.shape
    E = edge_index.shape[1]
    sidx = edge_index[0]
    didx = edge_index[1]
    return _make_sc_kernel(E, N, D)(h_src, h_dst, sidx, didx)
